# baseline (speedup 1.0000x reference)
import functools

import jax
import jax.numpy as jnp
from jax import lax
from jax.experimental import pallas as pl
from jax.experimental.pallas import tpu as pltpu
from jax.experimental.pallas import tpu_sc as plsc

_VOCAB = 1000000
_EMBED_DIM = 32
_BATCH = 16384
_FIELDS = 26
_N_TOTAL = _BATCH * _FIELDS
_NC, _NS = 2, 16
_NW = _NC * _NS

_mesh = plsc.VectorSubcoreMesh(core_axis_name="c", subcore_axis_name="s")

_SB = 768
_NBLK = _VOCAB // _SB
_A_ITERS = -(-_NBLK // _NW)


@functools.partial(
    pl.kernel,
    mesh=_mesh,
    out_type=jax.ShapeDtypeStruct((_VOCAB // 4, 128), jnp.float32),
    scratch_types=[
        pltpu.VMEM((32, _SB), jnp.float32),
        pltpu.VMEM((_SB // 4, 128), jnp.float32),
    ],
    compiler_params=pltpu.CompilerParams(use_tc_tiling_on_sc=True,
                                         needs_layout_passes=False),
)
def _repack(emb_t_hbm, tail_hbm, tlin_hbm, tin, tout):
    wid = lax.axis_index("s") * _NC + lax.axis_index("c")
    iota = lax.iota(jnp.int32, 16)
    rows_even = iota
    rows_odd = iota + 16

    def block(k, carry):
        blk = wid + _NW * k

        @pl.when(blk < _NBLK)
        def _():
            v0 = pl.multiple_of(blk * _SB, 128)
            pltpu.sync_copy(emb_t_hbm.at[pl.ds(0, 32), pl.ds(v0, _SB)], tin)

            def prow(p_lo, c2):
                for h in range(8):
                    rows = rows_even if (h & 1) == 0 else rows_odd
                    cols = jnp.full((16,), 4 * p_lo + (h >> 1), jnp.int32)
                    val = plsc.load_gather(tin, [rows, cols])
                    tout[p_lo, pl.ds(16 * h, 16)] = val
                return c2

            lax.fori_loop(0, _SB // 4, prow, 0)
            p0 = pl.multiple_of(blk * (_SB // 4), 8)
            pltpu.sync_copy(tout, tlin_hbm.at[pl.ds(p0, _SB // 4)])
        return carry

    lax.fori_loop(0, _A_ITERS, block, 0)

    @pl.when(wid == 0)
    def _():
        pltpu.sync_copy(tail_hbm,
                        tlin_hbm.at[pl.ds(_VOCAB // 4 - 16, 16)])


_PER_W = _N_TOTAL // _NW
_CHUNK = 1664
_N_CHUNKS = _PER_W // _CHUNK


@functools.partial(
    pl.kernel,
    mesh=_mesh,
    out_type=jax.ShapeDtypeStruct((_N_TOTAL, _EMBED_DIM), jnp.float32),
    scratch_types=[
        pltpu.VMEM((_PER_W,), jnp.int32),
        pltpu.VMEM((2, _CHUNK, _EMBED_DIM), jnp.float32),
        pltpu.SemaphoreType.DMA,
        pltpu.SemaphoreType.DMA,
        pltpu.SemaphoreType.DMA,
        pltpu.SemaphoreType.DMA,
    ],
    compiler_params=pltpu.CompilerParams(use_tc_tiling_on_sc=False),
)
def _gather(idx_hbm, table_hbm, out_hbm, idx_v, rows_v, sg0, sg1, ss0, ss1):
    wid = lax.axis_index("s") * _NC + lax.axis_index("c")
    base = wid * _PER_W
    sem_g = (sg0, sg1)
    sem_s = (ss0, ss1)

    pltpu.sync_copy(idx_hbm.at[pl.ds(base, _PER_W)], idx_v)

    gathers = [None] * _N_CHUNKS
    stores = [None] * _N_CHUNKS

    def start_gather(c):
        slot = c & 1
        g = pltpu.make_async_copy(
            table_hbm.at[idx_v.at[pl.ds(c * _CHUNK, _CHUNK)]],
            rows_v.at[slot], sem_g[slot])
        g.start()
        gathers[c] = g

    def start_store(c):
        slot = c & 1
        s = pltpu.make_async_copy(
            rows_v.at[slot],
            out_hbm.at[pl.ds(base + c * _CHUNK, _CHUNK)], sem_s[slot])
        s.start()
        stores[c] = s

    for c in range(_N_CHUNKS):
        if c >= 2:
            stores[c - 2].wait()
        start_gather(c)
        if c >= 1:
            gathers[c - 1].wait()
            start_store(c - 1)
    gathers[_N_CHUNKS - 1].wait()
    start_store(_N_CHUNKS - 1)
    stores[_N_CHUNKS - 2].wait()
    stores[_N_CHUNKS - 1].wait()


_B_PER_W = _BATCH // _NW


@functools.partial(
    pl.kernel,
    mesh=_mesh,
    out_type=jax.ShapeDtypeStruct((_FIELDS, _EMBED_DIM, _BATCH),
                                  jnp.float32),
    scratch_types=[
        pltpu.VMEM((_B_PER_W // 4, 128), jnp.float32),
        pltpu.VMEM((_EMBED_DIM, _B_PER_W), jnp.float32),
    ],
    compiler_params=pltpu.CompilerParams(use_tc_tiling_on_sc=True,
                                         needs_layout_passes=False),
)
def _format(lin_hbm, o_hbm, fin, fout):
    wid = lax.axis_index("s") * _NC + lax.axis_index("c")
    b0 = pl.multiple_of(wid * _B_PER_W, 128)
    iota = lax.iota(jnp.int32, 16)
    qbase = iota >> 2
    cbase = (iota & 3) << 5

    def field(f, carry):
        r0 = pl.multiple_of(f * (_BATCH // 4) + wid * (_B_PER_W // 4), 8)
        pltpu.sync_copy(lin_hbm.at[pl.ds(r0, _B_PER_W // 4)], fin)

        def hloop(h, c2):
            rows = qbase + 4 * h

            def dloop(d, c3):
                cols = cbase + d
                val = plsc.load_gather(fin, [rows, cols])
                fout[d, pl.ds(16 * h, 16)] = val
                return c3

            lax.fori_loop(0, _EMBED_DIM, dloop, 0)
            return c2

        lax.fori_loop(0, _B_PER_W // 16, hloop, 0)
        pltpu.sync_copy(fout,
                        o_hbm.at[f, pl.ds(0, _EMBED_DIM), pl.ds(b0, _B_PER_W)])
        return carry

    lax.fori_loop(0, _FIELDS, field, 0)


def kernel(inputs, embedding):
    emb_t = embedding.T
    tail = lax.slice(embedding, (_VOCAB - 64, 0),
                     (_VOCAB, _EMBED_DIM)).reshape(16, 128)
    tlin = _repack(emb_t, tail)
    table = tlin.reshape(_VOCAB, _EMBED_DIM)
    flat_idx = inputs.T.reshape(-1)
    out2d = _gather(flat_idx, table)
    out128 = out2d.reshape(_N_TOTAL // 4, 128)
    o = _format(out128)
    return o.transpose(2, 0, 1)
